# Initial kernel scaffold; baseline (speedup 1.0000x reference)
import jax
import jax.numpy as jnp
from jax.experimental import pallas as pl


def kernel(x, tables):
    raise NotImplementedError("write your pallas kernel here")


# speedup vs baseline: 3.8665x; 3.8665x over previous
import jax
import jax.numpy as jnp
from jax import lax
from jax.experimental import pallas as pl
from jax.experimental.pallas import tpu as pltpu
from jax.experimental.pallas import tpu_sc as plsc

BATCH = 16384
N_CAT = 26
N_CONT = 13
VOCAB = 100000
EMBED_DIM = 32
N_FEAT = N_CAT + N_CONT
OUT_W = N_CAT * EMBED_DIM + N_CONT

NC = 2
NS = 16
NW = NC * NS
PAIRS = N_CAT * EMBED_DIM
PPW = PAIRS // NW
BC = 4096
NCHUNK = BATCH // BC


def _body(xt_hbm, tt_hbm, outt_hbm, row_v, idx_v, xf_v, stage_v, sem):
    wid = lax.axis_index("s") * NC + lax.axis_index("c")
    p0 = wid * PPW

    def load_idx(c):
        def chunk(k, carry):
            pltpu.sync_copy(xt_hbm.at[c, pl.ds(k * BC, BC)], xf_v)
            for i in range(BC // 16):
                v = xf_v[pl.ds(16 * i, 16)]
                idx_v[pl.ds(k * BC + 16 * i, 16)] = v.astype(jnp.int32)
            return carry
        lax.fori_loop(0, NCHUNK, chunk, 0)

    def pair(t, carry):
        p = p0 + t
        c = p // EMBED_DIM
        e = p % EMBED_DIM

        @pl.when((t == 0) | (e == 0))
        def _():
            load_idx(c)

        pltpu.sync_copy(tt_hbm.at[c, e, :], row_v)

        def chunk(k, carry2):
            for i in range(BC // 16):
                iv = idx_v[pl.ds(k * BC + 16 * i, 16)]
                stage_v[pl.ds(16 * i, 16)] = plsc.load_gather(row_v, [iv])
            pltpu.sync_copy(stage_v, outt_hbm.at[p, pl.ds(k * BC, BC)])
            return carry2
        lax.fori_loop(0, NCHUNK, chunk, 0)
        return carry

    lax.fori_loop(0, PPW, pair, 0)

    @pl.when(wid < N_CONT)
    def _():
        def chunk(k, carry):
            pltpu.sync_copy(xt_hbm.at[N_CAT + wid, pl.ds(k * BC, BC)], xf_v)
            pltpu.sync_copy(xf_v, outt_hbm.at[PAIRS + wid, pl.ds(k * BC, BC)])
            return carry
        lax.fori_loop(0, NCHUNK, chunk, 0)


def kernel(x, tables):
    xt = x.T
    tt = jnp.transpose(tables, (0, 2, 1))
    f = pl.kernel(
        _body,
        out_type=jax.ShapeDtypeStruct((OUT_W, BATCH), jnp.float32),
        mesh=plsc.VectorSubcoreMesh(core_axis_name="c", subcore_axis_name="s"),
        compiler_params=pltpu.CompilerParams(needs_layout_passes=False),
        scratch_types=[
            pltpu.VMEM((VOCAB,), jnp.float32),
            pltpu.VMEM((BATCH,), jnp.int32),
            pltpu.VMEM((BC,), jnp.float32),
            pltpu.VMEM((BC,), jnp.float32),
            pltpu.SemaphoreType.DMA,
        ],
    )
    outt = f(xt, tt)
    return outt.T


# speedup vs baseline: 5.6159x; 1.4524x over previous
import jax
import jax.numpy as jnp
from jax import lax
from jax.experimental import pallas as pl
from jax.experimental.pallas import tpu as pltpu
from jax.experimental.pallas import tpu_sc as plsc

BATCH = 16384
N_CAT = 26
N_CONT = 13
VOCAB = 100000
EMBED_DIM = 32
N_FEAT = N_CAT + N_CONT
OUT_W = N_CAT * EMBED_DIM + N_CONT

NC = 2
NS = 16
NW = NC * NS
PAIRS = N_CAT * EMBED_DIM
PPW = PAIRS // NW
BC = 4096
NCHUNK = BATCH // BC

HALF = 49920
TAIL0 = 2 * HALF
TAILPAD = 256
ROWPAD = TAIL0 + TAILPAD


def _body(xt_hbm, tt_hbm, tail_hbm, outt_hbm,
          row_v, idx_v, xf_v, st0_v, st1_v, sem_row, sem_out):
    wid = lax.axis_index("s") * NC + lax.axis_index("c")
    p0 = wid * PPW

    def load_idx(c):
        def chunk(k, carry):
            pltpu.sync_copy(xt_hbm.at[c, pl.ds(k * BC, BC)], xf_v)
            for i in range(BC // 16):
                v = xf_v[pl.ds(16 * i, 16)]
                idx_v[pl.ds(k * BC + 16 * i, 16)] = v.astype(jnp.int32)
            return carry
        lax.fori_loop(0, NCHUNK, chunk, 0)

    def pair(t, carry):
        p = p0 + t
        c = p // EMBED_DIM
        e = p % EMBED_DIM

        @pl.when((t == 0) | (e == 0))
        def _():
            load_idx(c)

        row_cps = [
            pltpu.async_copy(tt_hbm.at[c, e, pl.ds(0, HALF)],
                             row_v.at[pl.ds(0, HALF)], sem_row),
            pltpu.async_copy(tt_hbm.at[c, e, pl.ds(HALF, HALF)],
                             row_v.at[pl.ds(HALF, HALF)], sem_row),
            pltpu.async_copy(tail_hbm.at[c, e, :],
                             row_v.at[pl.ds(TAIL0, TAILPAD)], sem_row),
        ]
        for cp in row_cps:
            cp.wait()

        out_cps = []
        for k in range(NCHUNK):
            stage = st0_v if k % 2 == 0 else st1_v
            if k >= 2:
                out_cps[k - 2].wait()

            def m_loop(m, carry2, k=k, stage=stage):
                for i in range(16):
                    off = m * 256 + 16 * i
                    iv = idx_v[pl.ds(k * BC + off, 16)]
                    stage[pl.ds(off, 16)] = plsc.load_gather(row_v, [iv])
                return carry2
            lax.fori_loop(0, BC // 256, m_loop, 0)
            out_cps.append(pltpu.async_copy(
                stage, outt_hbm.at[p, pl.ds(k * BC, BC)], sem_out))
        out_cps[NCHUNK - 2].wait()
        out_cps[NCHUNK - 1].wait()
        return carry

    lax.fori_loop(0, PPW, pair, 0)

    @pl.when(wid < N_CONT)
    def _():
        def chunk(k, carry):
            pltpu.sync_copy(xt_hbm.at[N_CAT + wid, pl.ds(k * BC, BC)], xf_v)
            pltpu.sync_copy(xf_v, outt_hbm.at[PAIRS + wid, pl.ds(k * BC, BC)])
            return carry
        lax.fori_loop(0, NCHUNK, chunk, 0)


def kernel(x, tables):
    xt = x.T
    tt = jnp.transpose(tables, (0, 2, 1))
    tail = jnp.pad(jnp.transpose(tables[:, TAIL0:, :], (0, 2, 1)),
                   ((0, 0), (0, 0), (0, TAILPAD - (VOCAB - TAIL0))))
    f = pl.kernel(
        _body,
        out_type=jax.ShapeDtypeStruct((OUT_W, BATCH), jnp.float32),
        mesh=plsc.VectorSubcoreMesh(core_axis_name="c", subcore_axis_name="s"),
        compiler_params=pltpu.CompilerParams(needs_layout_passes=False),
        scratch_types=[
            pltpu.VMEM((ROWPAD,), jnp.float32),
            pltpu.VMEM((BATCH,), jnp.int32),
            pltpu.VMEM((BC,), jnp.float32),
            pltpu.VMEM((BC,), jnp.float32),
            pltpu.VMEM((BC,), jnp.float32),
            pltpu.SemaphoreType.DMA,
            pltpu.SemaphoreType.DMA,
        ],
    )
    outt = f(xt, tt, tail)
    return outt.T


# speedup vs baseline: 5.6231x; 1.0013x over previous
import jax
import jax.numpy as jnp
from jax import lax
from jax.experimental import pallas as pl
from jax.experimental.pallas import tpu as pltpu
from jax.experimental.pallas import tpu_sc as plsc

BATCH = 16384
N_CAT = 26
N_CONT = 13
VOCAB = 100000
EMBED_DIM = 32
N_FEAT = N_CAT + N_CONT
OUT_W = N_CAT * EMBED_DIM + N_CONT

NC = 2
NS = 16
NW = NC * NS
PAIRS = N_CAT * EMBED_DIM
PPW = PAIRS // NW
BC = 4096
NCHUNK = BATCH // BC

HALF = 49920
TAIL0 = 2 * HALF
TAILPAD = 256
ROWPAD = TAIL0 + TAILPAD


def _body(xt_hbm, tt_hbm, tail_hbm, outt_hbm,
          row_v, idx_v, xf_v, st0_v, st1_v, sem_row, sem_out):
    wid = lax.axis_index("s") * NC + lax.axis_index("c")
    p0 = wid * PPW

    def load_idx(c):
        def chunk(k, carry):
            pltpu.sync_copy(xt_hbm.at[c, pl.ds(k * BC, BC)], xf_v)
            for i in range(BC // 16):
                v = xf_v[pl.ds(16 * i, 16)]
                idx_v[pl.ds(k * BC + 16 * i, 16)] = v.astype(jnp.int32)
            return carry
        lax.fori_loop(0, NCHUNK, chunk, 0)

    def pair(t, carry):
        p = p0 + t
        c = p // EMBED_DIM
        e = p % EMBED_DIM

        @pl.when((t == 0) | (e == 0))
        def _():
            load_idx(c)

        QTR = HALF // 2
        row_cps = [
            pltpu.async_copy(tt_hbm.at[c, e, pl.ds(q * QTR, QTR)],
                             row_v.at[pl.ds(q * QTR, QTR)], sem_row)
            for q in range(4)
        ] + [
            pltpu.async_copy(tail_hbm.at[c, e, :],
                             row_v.at[pl.ds(TAIL0, TAILPAD)], sem_row),
        ]
        for cp in row_cps:
            cp.wait()

        out_cps = []
        for k in range(NCHUNK):
            stage = st0_v if k % 2 == 0 else st1_v
            if k >= 2:
                out_cps[k - 2].wait()

            def m_loop(m, carry2, k=k, stage=stage):
                for i in range(16):
                    off = m * 256 + 16 * i
                    iv = idx_v[pl.ds(k * BC + off, 16)]
                    stage[pl.ds(off, 16)] = plsc.load_gather(row_v, [iv])
                return carry2
            lax.fori_loop(0, BC // 256, m_loop, 0)
            out_cps.append(pltpu.async_copy(
                stage, outt_hbm.at[p, pl.ds(k * BC, BC)], sem_out))
        out_cps[NCHUNK - 2].wait()
        out_cps[NCHUNK - 1].wait()
        return carry

    lax.fori_loop(0, PPW, pair, 0)

    @pl.when(wid < N_CONT)
    def _():
        def chunk(k, carry):
            pltpu.sync_copy(xt_hbm.at[N_CAT + wid, pl.ds(k * BC, BC)], xf_v)
            pltpu.sync_copy(xf_v, outt_hbm.at[PAIRS + wid, pl.ds(k * BC, BC)])
            return carry
        lax.fori_loop(0, NCHUNK, chunk, 0)


def kernel(x, tables):
    xt = x.T
    tt = jnp.transpose(tables, (0, 2, 1))
    tail = jnp.pad(jnp.transpose(tables[:, TAIL0:, :], (0, 2, 1)),
                   ((0, 0), (0, 0), (0, TAILPAD - (VOCAB - TAIL0))))
    f = pl.kernel(
        _body,
        out_type=jax.ShapeDtypeStruct((OUT_W, BATCH), jnp.float32),
        mesh=plsc.VectorSubcoreMesh(core_axis_name="c", subcore_axis_name="s"),
        compiler_params=pltpu.CompilerParams(needs_layout_passes=False),
        scratch_types=[
            pltpu.VMEM((ROWPAD,), jnp.float32),
            pltpu.VMEM((BATCH,), jnp.int32),
            pltpu.VMEM((BC,), jnp.float32),
            pltpu.VMEM((BC,), jnp.float32),
            pltpu.VMEM((BC,), jnp.float32),
            pltpu.SemaphoreType.DMA,
            pltpu.SemaphoreType.DMA,
        ],
    )
    outt = f(xt, tt, tail)
    return outt.T
